# baseline (speedup 1.0000x reference)
import functools

import jax
import jax.numpy as jnp
from jax import lax
from jax.experimental import pallas as pl
from jax.experimental.pallas import tpu as pltpu
from jax.experimental.pallas import tpu_sc as plsc

_NC, _NS = 2, 16
_NW = _NC * _NS
_L = 16
_RING = 4
_AHEAD = 2


def kernel(x, perm):
    B, C, H, W = x.shape
    R = B * C
    Q = R * 2
    Hh = H // 2
    n = Q // _NW

    x3 = x.reshape(Q, Hh, W)
    src = (jnp.arange(B, dtype=jnp.int32)[:, None] * C
           + perm.astype(jnp.int32)[None, :]).reshape(R)
    src2 = (src[:, None] * 2
            + jnp.arange(2, dtype=jnp.int32)[None, :]).reshape(Q)
    src_rep = jnp.broadcast_to(src2[:, None], (Q, _L)).reshape(Q * _L)

    @functools.partial(
        pl.kernel,
        mesh=plsc.VectorSubcoreMesh(core_axis_name="c", subcore_axis_name="s"),
        out_type=jax.ShapeDtypeStruct((Q, Hh, W), jnp.float32),
        scratch_types=[
            pltpu.VMEM((n * _L,), jnp.int32),
            pltpu.VMEM((2, Hh, W), jnp.float32),
            pltpu.VMEM_SHARED((_NS, 2, Hh, W), jnp.float32),
            [pltpu.SemaphoreType.DMA] * _RING,
            [pltpu.SemaphoreType.DMA] * _RING,
        ],
    )
    def sc_gather(x_hbm, src_hbm, out_hbm, idx_v, tbuf, sbuf, gsems, ssems):
        sid = lax.axis_index("s")
        wid = sid * _NC + lax.axis_index("c")
        base = wid * n
        pltpu.sync_copy(src_hbm.at[pl.ds(base * _L, n * _L)], idx_v)

        def idx_at(p):
            return idx_v[pl.ds(p * _L, _L)][0]

        bufs = [sbuf.at[sid, 0], tbuf.at[0], sbuf.at[sid, 1], tbuf.at[1]]

        for t in range(_AHEAD):
            pltpu.async_copy(x_hbm.at[idx_at(t)], bufs[t], gsems[t])

        @pl.loop(0, n, step=_RING)
        def _steps(j):
            for b in range(_RING):
                t = j + b
                pltpu.make_async_copy(x_hbm.at[0], bufs[b], gsems[b]).wait()
                pltpu.async_copy(bufs[b], out_hbm.at[base + t], ssems[b])
                b2 = (b + _AHEAD) % _RING

                @pl.when(t >= _AHEAD)
                def _retire_scatter():
                    pltpu.make_async_copy(
                        x_hbm.at[0], bufs[b2], ssems[b2]).wait()

                @pl.when(t + _AHEAD < n)
                def _issue_gather():
                    pltpu.async_copy(
                        x_hbm.at[idx_at(t + _AHEAD)], bufs[b2], gsems[b2])

        for k in range(_AHEAD):
            b2 = (n - _AHEAD + k) % _RING
            pltpu.make_async_copy(x_hbm.at[0], bufs[b2], ssems[b2]).wait()

    return sc_gather(x3, src_rep).reshape(B, C, H, W)


# speedup vs baseline: 1.0322x; 1.0322x over previous
import functools

import jax
import jax.numpy as jnp
from jax import lax
from jax.experimental import pallas as pl
from jax.experimental.pallas import tpu as pltpu
from jax.experimental.pallas import tpu_sc as plsc

_NC, _NS = 2, 16
_NW = _NC * _NS
_L = 16


def kernel(x, perm):
    B, C, H, W = x.shape
    R = B * C
    rpw = R // _NW
    gpw = rpw // _L

    x3 = x.reshape(R, H, W)
    src = (jnp.arange(B, dtype=jnp.int32)[:, None] * C
           + perm.astype(jnp.int32)[None, :]).reshape(R)

    @functools.partial(
        pl.kernel,
        mesh=plsc.VectorSubcoreMesh(core_axis_name="c", subcore_axis_name="s"),
        out_type=jax.ShapeDtypeStruct((R, H, W), jnp.float32),
        scratch_types=[
            pltpu.VMEM((rpw,), jnp.int32),
            pltpu.VMEM_SHARED((_NS, 2, H, W), jnp.float32),
            pltpu.SemaphoreType.DMA,
            pltpu.SemaphoreType.DMA,
        ],
    )
    def sc_gather(x_hbm, src_hbm, out_hbm, idx_v, buf_v, sem0, sem1):
        sid = lax.axis_index("s")
        wid = sid * _NC + lax.axis_index("c")
        base = wid * rpw
        pltpu.sync_copy(src_hbm.at[pl.ds(base, rpw)], idx_v)
        sems = (sem0, sem1)

        c0 = idx_v[pl.ds(0, _L)]
        for b in range(2):
            pltpu.async_copy(x_hbm.at[c0[b]], buf_v.at[sid, b], sems[b])

        @pl.loop(0, gpw)
        def _groups(g):
            goff = g * _L
            chunk = idx_v[pl.ds(goff, _L)]
            noff = jnp.minimum(goff + _L, (gpw - 1) * _L)
            nchunk = idx_v[pl.ds(noff, _L)]
            for k in range(_L):
                b = k % 2
                row = goff + k
                pltpu.make_async_copy(
                    x_hbm.at[0], buf_v.at[sid, b], sems[b]).wait()
                pltpu.sync_copy(buf_v.at[sid, b], out_hbm.at[base + row])
                nxt = chunk[k + 2] if k + 2 < _L else nchunk[k + 2 - _L]

                @pl.when(row + 2 < rpw)
                def _issue_next():
                    pltpu.async_copy(x_hbm.at[nxt], buf_v.at[sid, b], sems[b])

    return sc_gather(x3, src).reshape(B, C, H, W)
